# baseline (speedup 1.0000x reference)
import functools
import jax
import jax.numpy as jnp
from jax import lax
from jax.experimental import pallas as pl
from jax.experimental.pallas import tpu as pltpu
from jax.experimental.pallas import tpu_sc as plsc

N = 10000
IN = 256
OUT = 256
H = 2
NEG = 0.2

NPAD = 10240
DUMMY = NPAD - 1
EB = 2048
NC, NS, L = 2, 16, 16
NW = NC * NS
NPW = NPAD // NW
BA = 512
BD = 256


def _dense_body(x_ref, w_ref, at_ref, et_ref, emb_ref, xw_ref, asc_ref):
    xwb = jnp.dot(x_ref[...], w_ref[...], preferred_element_type=jnp.float32)
    xw_ref[...] = xwb
    hi = lax.Precision.HIGHEST
    asc_ref[...] = (
        jnp.dot(xwb, at_ref[...], preferred_element_type=jnp.float32,
                precision=hi)
        + jnp.dot(emb_ref[...], et_ref[...], preferred_element_type=jnp.float32,
                  precision=hi)
    )


def _leaky_exp(a):
    return jnp.exp(jnp.where(a >= 0, a, NEG * a))


def _make_sc_b1(epad):
    epw = epad // NW
    mesh = plsc.VectorSubcoreMesh(
        core_axis_name="c", subcore_axis_name="s", num_cores=NC, num_subcores=NS
    )
    f32 = jnp.float32

    @functools.partial(
        pl.kernel,
        out_type=(
            jax.ShapeDtypeStruct((epad,), f32),
            jax.ShapeDtypeStruct((epad,), f32),
            jax.ShapeDtypeStruct((NPAD,), f32),
            jax.ShapeDtypeStruct((NPAD,), f32),
            jax.ShapeDtypeStruct((NW, NPAD), f32),
            jax.ShapeDtypeStruct((NW, NPAD), f32),
        ),
        mesh=mesh,
        scratch_types=[
            pltpu.VMEM((epw,), jnp.int32),
            pltpu.VMEM((epw,), jnp.int32),
            pltpu.VMEM((NPAD,), f32),
            pltpu.VMEM((NPAD,), f32),
            pltpu.VMEM((NPAD,), f32),
            pltpu.VMEM((NPAD,), f32),
            pltpu.VMEM((NPAD,), f32),
            pltpu.VMEM((NPAD,), f32),
            pltpu.VMEM((epw,), f32),
            pltpu.VMEM((epw,), f32),
            pltpu.VMEM((NPW,), f32),
            pltpu.VMEM((NPW,), f32),
        ],
        compiler_params=pltpu.CompilerParams(needs_layout_passes=False),
    )
    def sc_b1(src_h, dst_h, ad0_h, ad1_h, as0_h, as1_h,
              ex0_h, ex1_h, sx0_h, sx1_h, p0_h, p1_h,
              v_src, v_dst, v_ad0, v_ad1, v_as0, v_as1,
              v_es0, v_es1, v_ex0, v_ex1, v_sx0, v_sx1):
        wid = lax.axis_index("s") * NC + lax.axis_index("c")
        ebase = wid * epw
        nbase = wid * NPW

        pltpu.sync_copy(src_h.at[pl.ds(ebase, epw)], v_src)
        pltpu.sync_copy(dst_h.at[pl.ds(ebase, epw)], v_dst)
        pltpu.sync_copy(ad0_h, v_ad0)
        pltpu.sync_copy(ad1_h, v_ad1)
        pltpu.sync_copy(as0_h, v_as0)
        pltpu.sync_copy(as1_h, v_as1)

        zeros16 = jnp.zeros((L,), f32)

        def _zero(i, _):
            o = pl.multiple_of(i * L, L)
            v_es0[pl.ds(o, L)] = zeros16
            v_es1[pl.ds(o, L)] = zeros16
            return 0

        lax.fori_loop(0, NPAD // L, _zero, 0)

        def _self(i, _):
            o = pl.multiple_of(i * L, L)
            no = pl.multiple_of(nbase + i * L, L)
            e0 = _leaky_exp(v_ad0[pl.ds(no, L)] + v_as0[pl.ds(no, L)])
            e1 = _leaky_exp(v_ad1[pl.ds(no, L)] + v_as1[pl.ds(no, L)])
            v_sx0[pl.ds(o, L)] = e0
            v_sx1[pl.ds(o, L)] = e1
            v_es0[pl.ds(no, L)] = v_es0[pl.ds(no, L)] + e0
            v_es1[pl.ds(no, L)] = v_es1[pl.ds(no, L)] + e1
            return 0

        lax.fori_loop(0, NPW // L, _self, 0)

        def _edges(i, _):
            o = pl.multiple_of(i * L, L)
            vs = v_src[pl.ds(o, L)]
            vd = v_dst[pl.ds(o, L)]
            a0 = plsc.load_gather(v_ad0, [vd]) + plsc.load_gather(v_as0, [vs])
            a1 = plsc.load_gather(v_ad1, [vd]) + plsc.load_gather(v_as1, [vs])
            e0 = _leaky_exp(a0)
            e1 = _leaky_exp(a1)
            v_ex0[pl.ds(o, L)] = e0
            v_ex1[pl.ds(o, L)] = e1
            plsc.addupdate_scatter(v_es0, [vd], e0)
            plsc.addupdate_scatter(v_es1, [vd], e1)
            return 0

        lax.fori_loop(0, epw // L, _edges, 0)

        pltpu.sync_copy(v_ex0, ex0_h.at[pl.ds(ebase, epw)])
        pltpu.sync_copy(v_ex1, ex1_h.at[pl.ds(ebase, epw)])
        pltpu.sync_copy(v_sx0, sx0_h.at[pl.ds(nbase, NPW)])
        pltpu.sync_copy(v_sx1, sx1_h.at[pl.ds(nbase, NPW)])
        pltpu.sync_copy(v_es0, p0_h.at[wid])
        pltpu.sync_copy(v_es1, p1_h.at[wid])

    return sc_b1


def _reduce_body(p0_ref, p1_ref, e0_ref, e1_ref):
    e0_ref[...] = jnp.broadcast_to(
        jnp.sum(p0_ref[...], axis=0, keepdims=True), (8, NPAD)
    )
    e1_ref[...] = jnp.broadcast_to(
        jnp.sum(p1_ref[...], axis=0, keepdims=True), (8, NPAD)
    )


def _make_sc_b3(epad):
    epw = epad // NW
    mesh = plsc.VectorSubcoreMesh(
        core_axis_name="c", subcore_axis_name="s", num_cores=NC, num_subcores=NS
    )
    f32 = jnp.float32

    @functools.partial(
        pl.kernel,
        out_type=(
            jax.ShapeDtypeStruct((epad,), f32),
            jax.ShapeDtypeStruct((epad,), f32),
            jax.ShapeDtypeStruct((NPAD,), f32),
            jax.ShapeDtypeStruct((NPAD,), f32),
        ),
        mesh=mesh,
        scratch_types=[
            pltpu.VMEM((epw,), jnp.int32),
            pltpu.VMEM((NPAD,), f32),
            pltpu.VMEM((NPAD,), f32),
            pltpu.VMEM((epw,), f32),
            pltpu.VMEM((epw,), f32),
            pltpu.VMEM((NPW,), f32),
            pltpu.VMEM((NPW,), f32),
        ],
        compiler_params=pltpu.CompilerParams(needs_layout_passes=False),
    )
    def sc_b3(dst_h, ex0_h, ex1_h, sx0_h, sx1_h, es0_h, es1_h,
              w0_h, w1_h, ws0_h, ws1_h,
              v_dst, v_es0, v_es1, v_ex0, v_ex1, v_sx0, v_sx1):
        wid = lax.axis_index("s") * NC + lax.axis_index("c")
        ebase = wid * epw
        nbase = wid * NPW

        pltpu.sync_copy(dst_h.at[pl.ds(ebase, epw)], v_dst)
        pltpu.sync_copy(ex0_h.at[pl.ds(ebase, epw)], v_ex0)
        pltpu.sync_copy(ex1_h.at[pl.ds(ebase, epw)], v_ex1)
        pltpu.sync_copy(sx0_h.at[pl.ds(nbase, NPW)], v_sx0)
        pltpu.sync_copy(sx1_h.at[pl.ds(nbase, NPW)], v_sx1)
        pltpu.sync_copy(es0_h, v_es0)
        pltpu.sync_copy(es1_h, v_es1)

        def _norm(i, _):
            o = pl.multiple_of(i * L, L)
            vd = v_dst[pl.ds(o, L)]
            v_ex0[pl.ds(o, L)] = v_ex0[pl.ds(o, L)] / plsc.load_gather(v_es0, [vd])
            v_ex1[pl.ds(o, L)] = v_ex1[pl.ds(o, L)] / plsc.load_gather(v_es1, [vd])
            return 0

        lax.fori_loop(0, epw // L, _norm, 0)

        def _normself(i, _):
            o = pl.multiple_of(i * L, L)
            no = pl.multiple_of(nbase + i * L, L)
            v_sx0[pl.ds(o, L)] = v_sx0[pl.ds(o, L)] / v_es0[pl.ds(no, L)]
            v_sx1[pl.ds(o, L)] = v_sx1[pl.ds(o, L)] / v_es1[pl.ds(no, L)]
            return 0

        lax.fori_loop(0, NPW // L, _normself, 0)

        pltpu.sync_copy(v_ex0, w0_h.at[pl.ds(ebase, epw)])
        pltpu.sync_copy(v_ex1, w1_h.at[pl.ds(ebase, epw)])
        pltpu.sync_copy(v_sx0, ws0_h.at[pl.ds(nbase, NPW)])
        pltpu.sync_copy(v_sx1, ws1_h.at[pl.ds(nbase, NPW)])

    return sc_b3


def _agg_body(src_ref, dst_ref, w0_ref, w1_ref, xw_ref, out_ref):
    @pl.when(pl.program_id(0) == 0)
    def _():
        out_ref[...] = jnp.zeros((NPAD, H * OUT), jnp.float32)

    U = 8

    def body(i, _):
        base = i * U
        ds_ = []
        msgs = []
        curs = []
        for k in range(U):
            s = src_ref[0, 0, base + k]
            d = dst_ref[0, 0, base + k]
            w0 = w0_ref[0, 0, base + k]
            w1 = w1_ref[0, 0, base + k]
            row = xw_ref[pl.ds(s, 1), :]
            wrow = jnp.concatenate(
                [jnp.full((1, OUT), w0, jnp.float32),
                 jnp.full((1, OUT), w1, jnp.float32)], axis=1)
            ds_.append(d)
            msgs.append(wrow * row)
            curs.append(out_ref[pl.ds(d, 1), :])
        vals = []
        for k in range(U):
            base_val = curs[k]
            for j in range(k):
                base_val = jnp.where(ds_[k] == ds_[j], vals[j], base_val)
            vals.append(base_val + msgs[k])
        for k in range(U):
            out_ref[pl.ds(ds_[k], 1), :] = vals[k]
        return 0

    lax.fori_loop(0, EB // U, body, 0)


def _selfbn_body(agg_ref, xw_ref, ws0_ref, ws1_ref, bias_ref,
                 pre_ref, psum_ref, psq_ref):
    pid = pl.program_id(0)

    @pl.when(pid == 0)
    def _():
        psum_ref[...] = jnp.zeros((8, H * OUT), jnp.float32)
        psq_ref[...] = jnp.zeros((8, H * OUT), jnp.float32)

    xwb = xw_ref[...]
    selfc = jnp.concatenate(
        [ws0_ref[:, 0:1] * xwb[:, :OUT], ws1_ref[:, 0:1] * xwb[:, OUT:]], axis=1)
    pre = agg_ref[...] + selfc + bias_ref[...]
    rid = pid * BD + lax.broadcasted_iota(jnp.int32, (BD, 1), 0)
    pre = pre * (rid < N).astype(jnp.float32)
    pre_ref[...] = pre
    psum_ref[0:1, :] = psum_ref[0:1, :] + jnp.sum(pre, axis=0, keepdims=True)
    psq_ref[0:1, :] = psq_ref[0:1, :] + jnp.sum(pre * pre, axis=0, keepdims=True)


def _bn_body(pre_ref, psum_ref, psq_ref, gamma_ref, beta_ref, y_ref):
    inv_n = 1.0 / N
    mean = psum_ref[0:1, :] * inv_n
    var = psq_ref[0:1, :] * inv_n - mean * mean
    rstd = lax.rsqrt(var + 1e-5)
    y = (pre_ref[...] - mean) * rstd * gamma_ref[...] + beta_ref[...]
    y_ref[...] = jnp.maximum(y, 0.0)


def kernel(x, edge_index, embedding, W, att_i, att_j, att_em_i, att_em_j,
           bias, gamma, beta):
    f32 = jnp.float32
    e = edge_index.shape[1]
    neb = -(-e // EB)
    epad = neb * EB
    src = edge_index[0].astype(jnp.int32)
    dst = edge_index[1].astype(jnp.int32)
    mask = src != dst
    dst_eff = jnp.where(mask, dst, DUMMY)
    src_p = jnp.concatenate([src, jnp.zeros((epad - e,), jnp.int32)])
    dst_p = jnp.concatenate([dst_eff, jnp.full((epad - e,), DUMMY, jnp.int32)])

    x_p = jnp.zeros((NPAD, IN), f32).at[:N].set(x)
    emb_p = jnp.zeros((NPAD, OUT), f32).at[:N].set(embedding)

    at = jnp.zeros((H * OUT, 128), f32)
    at = at.at[:OUT, 0].set(att_i[0, 0])
    at = at.at[OUT:, 1].set(att_i[0, 1])
    at = at.at[:OUT, 2].set(att_j[0, 0])
    at = at.at[OUT:, 3].set(att_j[0, 1])
    et = jnp.zeros((IN, 128), f32)
    et = et.at[:, 0].set(att_em_i[0, 0])
    et = et.at[:, 1].set(att_em_i[0, 1])
    et = et.at[:, 2].set(att_em_j[0, 0])
    et = et.at[:, 3].set(att_em_j[0, 1])

    xw, asc = pl.pallas_call(
        _dense_body,
        grid=(NPAD // BA,),
        in_specs=[
            pl.BlockSpec((BA, IN), lambda i: (i, 0)),
            pl.BlockSpec((IN, H * OUT), lambda i: (0, 0)),
            pl.BlockSpec((H * OUT, 128), lambda i: (0, 0)),
            pl.BlockSpec((IN, 128), lambda i: (0, 0)),
            pl.BlockSpec((BA, OUT), lambda i: (i, 0)),
        ],
        out_specs=[
            pl.BlockSpec((BA, H * OUT), lambda i: (i, 0)),
            pl.BlockSpec((BA, 128), lambda i: (i, 0)),
        ],
        out_shape=[
            jax.ShapeDtypeStruct((NPAD, H * OUT), f32),
            jax.ShapeDtypeStruct((NPAD, 128), f32),
        ],
    )(x_p, W.astype(f32), at, et, emb_p)

    ad0 = asc[:, 0]
    ad1 = asc[:, 1]
    as0 = asc[:, 2]
    as1 = asc[:, 3]

    ex0, ex1, sx0, sx1, p0, p1 = _make_sc_b1(epad)(
        src_p, dst_p, ad0, ad1, as0, as1)

    es0_8, es1_8 = pl.pallas_call(
        _reduce_body,
        grid=(1,),
        in_specs=[
            pl.BlockSpec((NW, NPAD), lambda i: (0, 0)),
            pl.BlockSpec((NW, NPAD), lambda i: (0, 0)),
        ],
        out_specs=[
            pl.BlockSpec((8, NPAD), lambda i: (0, 0)),
            pl.BlockSpec((8, NPAD), lambda i: (0, 0)),
        ],
        out_shape=[
            jax.ShapeDtypeStruct((8, NPAD), f32),
            jax.ShapeDtypeStruct((8, NPAD), f32),
        ],
    )(p0, p1)
    es0 = es0_8[0]
    es1 = es1_8[0]

    w0, w1, ws0, ws1 = _make_sc_b3(epad)(dst_p, ex0, ex1, sx0, sx1, es0, es1)

    smem = pltpu.MemorySpace.SMEM
    agg = pl.pallas_call(
        _agg_body,
        grid=(neb,),
        in_specs=[
            pl.BlockSpec((1, 1, EB), lambda i: (i, 0, 0), memory_space=smem),
            pl.BlockSpec((1, 1, EB), lambda i: (i, 0, 0), memory_space=smem),
            pl.BlockSpec((1, 1, EB), lambda i: (i, 0, 0), memory_space=smem),
            pl.BlockSpec((1, 1, EB), lambda i: (i, 0, 0), memory_space=smem),
            pl.BlockSpec((NPAD, H * OUT), lambda i: (0, 0)),
        ],
        out_specs=pl.BlockSpec((NPAD, H * OUT), lambda i: (0, 0)),
        out_shape=jax.ShapeDtypeStruct((NPAD, H * OUT), f32),
    )(src_p.reshape(neb, 1, EB), dst_p.reshape(neb, 1, EB),
      w0.reshape(neb, 1, EB), w1.reshape(neb, 1, EB), xw)

    ws0b = jnp.broadcast_to(ws0[:, None], (NPAD, 128))
    ws1b = jnp.broadcast_to(ws1[:, None], (NPAD, 128))
    bias2 = bias.astype(f32).reshape(1, H * OUT)
    pre, psum, psq = pl.pallas_call(
        _selfbn_body,
        grid=(NPAD // BD,),
        in_specs=[
            pl.BlockSpec((BD, H * OUT), lambda i: (i, 0)),
            pl.BlockSpec((BD, H * OUT), lambda i: (i, 0)),
            pl.BlockSpec((BD, 128), lambda i: (i, 0)),
            pl.BlockSpec((BD, 128), lambda i: (i, 0)),
            pl.BlockSpec((1, H * OUT), lambda i: (0, 0)),
        ],
        out_specs=[
            pl.BlockSpec((BD, H * OUT), lambda i: (i, 0)),
            pl.BlockSpec((8, H * OUT), lambda i: (0, 0)),
            pl.BlockSpec((8, H * OUT), lambda i: (0, 0)),
        ],
        out_shape=[
            jax.ShapeDtypeStruct((NPAD, H * OUT), f32),
            jax.ShapeDtypeStruct((8, H * OUT), f32),
            jax.ShapeDtypeStruct((8, H * OUT), f32),
        ],
    )(agg, xw, ws0b, ws1b, bias2)

    y = pl.pallas_call(
        _bn_body,
        grid=(NPAD // BD,),
        in_specs=[
            pl.BlockSpec((BD, H * OUT), lambda i: (i, 0)),
            pl.BlockSpec((8, H * OUT), lambda i: (0, 0)),
            pl.BlockSpec((8, H * OUT), lambda i: (0, 0)),
            pl.BlockSpec((1, H * OUT), lambda i: (0, 0)),
            pl.BlockSpec((1, H * OUT), lambda i: (0, 0)),
        ],
        out_specs=pl.BlockSpec((BD, H * OUT), lambda i: (i, 0)),
        out_shape=jax.ShapeDtypeStruct((NPAD, H * OUT), f32),
    )(pre, psum, psq, gamma.astype(f32).reshape(1, H * OUT),
      beta.astype(f32).reshape(1, H * OUT))

    return y[:N]


# speedup vs baseline: 1.2350x; 1.2350x over previous
import functools
import jax
import jax.numpy as jnp
from jax import lax
from jax.experimental import pallas as pl
from jax.experimental.pallas import tpu as pltpu
from jax.experimental.pallas import tpu_sc as plsc

N = 10000
IN = 256
OUT = 256
H = 2
NEG = 0.2

NPAD = 10240
DUMMY = NPAD - 1
EB = 2048
NC, NS, L = 2, 16, 16
NW = NC * NS
NPW = NPAD // NW
BA = 512
BD = 256


def _dense_body(x_ref, w_ref, at_ref, et_ref, emb_ref, xw_ref, asc_ref):
    xwb = jnp.dot(x_ref[...], w_ref[...], preferred_element_type=jnp.float32)
    xw_ref[...] = xwb
    hi = lax.Precision.HIGHEST
    asc_ref[...] = (
        jnp.dot(xwb, at_ref[...], preferred_element_type=jnp.float32,
                precision=hi)
        + jnp.dot(emb_ref[...], et_ref[...], preferred_element_type=jnp.float32,
                  precision=hi)
    )


def _leaky_exp(a):
    return jnp.exp(jnp.where(a >= 0, a, NEG * a))


def _make_sc_b1(epad):
    epw = epad // NW
    mesh = plsc.VectorSubcoreMesh(
        core_axis_name="c", subcore_axis_name="s", num_cores=NC, num_subcores=NS
    )
    f32 = jnp.float32

    @functools.partial(
        pl.kernel,
        out_type=(
            jax.ShapeDtypeStruct((epad,), f32),
            jax.ShapeDtypeStruct((epad,), f32),
            jax.ShapeDtypeStruct((NPAD,), f32),
            jax.ShapeDtypeStruct((NPAD,), f32),
            jax.ShapeDtypeStruct((NW, NPAD), f32),
            jax.ShapeDtypeStruct((NW, NPAD), f32),
        ),
        mesh=mesh,
        scratch_types=[
            pltpu.VMEM((epw,), jnp.int32),
            pltpu.VMEM((epw,), jnp.int32),
            pltpu.VMEM((NPAD,), f32),
            pltpu.VMEM((NPAD,), f32),
            pltpu.VMEM((NPAD,), f32),
            pltpu.VMEM((NPAD,), f32),
            pltpu.VMEM((NPAD,), f32),
            pltpu.VMEM((NPAD,), f32),
            pltpu.VMEM((epw,), f32),
            pltpu.VMEM((epw,), f32),
            pltpu.VMEM((NPW,), f32),
            pltpu.VMEM((NPW,), f32),
        ],
        compiler_params=pltpu.CompilerParams(needs_layout_passes=False),
    )
    def sc_b1(src_h, dst_h, ad0_h, ad1_h, as0_h, as1_h,
              ex0_h, ex1_h, sx0_h, sx1_h, p0_h, p1_h,
              v_src, v_dst, v_ad0, v_ad1, v_as0, v_as1,
              v_es0, v_es1, v_ex0, v_ex1, v_sx0, v_sx1):
        wid = lax.axis_index("s") * NC + lax.axis_index("c")
        ebase = wid * epw
        nbase = wid * NPW

        pltpu.sync_copy(src_h.at[pl.ds(ebase, epw)], v_src)
        pltpu.sync_copy(dst_h.at[pl.ds(ebase, epw)], v_dst)
        pltpu.sync_copy(ad0_h, v_ad0)
        pltpu.sync_copy(ad1_h, v_ad1)
        pltpu.sync_copy(as0_h, v_as0)
        pltpu.sync_copy(as1_h, v_as1)

        zeros16 = jnp.zeros((L,), f32)

        def _zero(i, _):
            o = pl.multiple_of(i * L, L)
            v_es0[pl.ds(o, L)] = zeros16
            v_es1[pl.ds(o, L)] = zeros16
            return 0

        lax.fori_loop(0, NPAD // L, _zero, 0)

        def _self(i, _):
            o = pl.multiple_of(i * L, L)
            no = pl.multiple_of(nbase + i * L, L)
            e0 = _leaky_exp(v_ad0[pl.ds(no, L)] + v_as0[pl.ds(no, L)])
            e1 = _leaky_exp(v_ad1[pl.ds(no, L)] + v_as1[pl.ds(no, L)])
            v_sx0[pl.ds(o, L)] = e0
            v_sx1[pl.ds(o, L)] = e1
            v_es0[pl.ds(no, L)] = v_es0[pl.ds(no, L)] + e0
            v_es1[pl.ds(no, L)] = v_es1[pl.ds(no, L)] + e1
            return 0

        lax.fori_loop(0, NPW // L, _self, 0)

        def _edges(i, _):
            o = pl.multiple_of(i * L, L)
            vs = v_src[pl.ds(o, L)]
            vd = v_dst[pl.ds(o, L)]
            a0 = plsc.load_gather(v_ad0, [vd]) + plsc.load_gather(v_as0, [vs])
            a1 = plsc.load_gather(v_ad1, [vd]) + plsc.load_gather(v_as1, [vs])
            e0 = _leaky_exp(a0)
            e1 = _leaky_exp(a1)
            v_ex0[pl.ds(o, L)] = e0
            v_ex1[pl.ds(o, L)] = e1
            plsc.addupdate_scatter(v_es0, [vd], e0)
            plsc.addupdate_scatter(v_es1, [vd], e1)
            return 0

        lax.fori_loop(0, epw // L, _edges, 0)

        pltpu.sync_copy(v_ex0, ex0_h.at[pl.ds(ebase, epw)])
        pltpu.sync_copy(v_ex1, ex1_h.at[pl.ds(ebase, epw)])
        pltpu.sync_copy(v_sx0, sx0_h.at[pl.ds(nbase, NPW)])
        pltpu.sync_copy(v_sx1, sx1_h.at[pl.ds(nbase, NPW)])
        pltpu.sync_copy(v_es0, p0_h.at[wid])
        pltpu.sync_copy(v_es1, p1_h.at[wid])

    return sc_b1


def _reduce_body(p0_ref, p1_ref, e0_ref, e1_ref):
    e0_ref[...] = jnp.broadcast_to(
        jnp.sum(p0_ref[...], axis=0, keepdims=True), (8, NPAD)
    )
    e1_ref[...] = jnp.broadcast_to(
        jnp.sum(p1_ref[...], axis=0, keepdims=True), (8, NPAD)
    )


def _make_sc_b3(epad):
    epw = epad // NW
    mesh = plsc.VectorSubcoreMesh(
        core_axis_name="c", subcore_axis_name="s", num_cores=NC, num_subcores=NS
    )
    f32 = jnp.float32

    @functools.partial(
        pl.kernel,
        out_type=(
            jax.ShapeDtypeStruct((epad,), f32),
            jax.ShapeDtypeStruct((epad,), f32),
            jax.ShapeDtypeStruct((NPAD,), f32),
            jax.ShapeDtypeStruct((NPAD,), f32),
        ),
        mesh=mesh,
        scratch_types=[
            pltpu.VMEM((epw,), jnp.int32),
            pltpu.VMEM((NPAD,), f32),
            pltpu.VMEM((NPAD,), f32),
            pltpu.VMEM((epw,), f32),
            pltpu.VMEM((epw,), f32),
            pltpu.VMEM((NPW,), f32),
            pltpu.VMEM((NPW,), f32),
        ],
        compiler_params=pltpu.CompilerParams(needs_layout_passes=False),
    )
    def sc_b3(dst_h, ex0_h, ex1_h, sx0_h, sx1_h, es0_h, es1_h,
              w0_h, w1_h, ws0_h, ws1_h,
              v_dst, v_es0, v_es1, v_ex0, v_ex1, v_sx0, v_sx1):
        wid = lax.axis_index("s") * NC + lax.axis_index("c")
        ebase = wid * epw
        nbase = wid * NPW

        pltpu.sync_copy(dst_h.at[pl.ds(ebase, epw)], v_dst)
        pltpu.sync_copy(ex0_h.at[pl.ds(ebase, epw)], v_ex0)
        pltpu.sync_copy(ex1_h.at[pl.ds(ebase, epw)], v_ex1)
        pltpu.sync_copy(sx0_h.at[pl.ds(nbase, NPW)], v_sx0)
        pltpu.sync_copy(sx1_h.at[pl.ds(nbase, NPW)], v_sx1)
        pltpu.sync_copy(es0_h, v_es0)
        pltpu.sync_copy(es1_h, v_es1)

        def _norm(i, _):
            o = pl.multiple_of(i * L, L)
            vd = v_dst[pl.ds(o, L)]
            v_ex0[pl.ds(o, L)] = v_ex0[pl.ds(o, L)] / plsc.load_gather(v_es0, [vd])
            v_ex1[pl.ds(o, L)] = v_ex1[pl.ds(o, L)] / plsc.load_gather(v_es1, [vd])
            return 0

        lax.fori_loop(0, epw // L, _norm, 0)

        def _normself(i, _):
            o = pl.multiple_of(i * L, L)
            no = pl.multiple_of(nbase + i * L, L)
            v_sx0[pl.ds(o, L)] = v_sx0[pl.ds(o, L)] / v_es0[pl.ds(no, L)]
            v_sx1[pl.ds(o, L)] = v_sx1[pl.ds(o, L)] / v_es1[pl.ds(no, L)]
            return 0

        lax.fori_loop(0, NPW // L, _normself, 0)

        pltpu.sync_copy(v_ex0, w0_h.at[pl.ds(ebase, epw)])
        pltpu.sync_copy(v_ex1, w1_h.at[pl.ds(ebase, epw)])
        pltpu.sync_copy(v_sx0, ws0_h.at[pl.ds(nbase, NPW)])
        pltpu.sync_copy(v_sx1, ws1_h.at[pl.ds(nbase, NPW)])

    return sc_b3


def _agg_body(src_ref, dst_ref, w0_ref, w1_ref, xw_ref, out_ref):
    @pl.when(pl.program_id(0) == 0)
    def _():
        out_ref[...] = jnp.zeros((NPAD, H * OUT), jnp.float32)

    U = 4

    def body(i, _):
        base = i * U
        ds_ = []
        msgs = []
        curs = []
        for k in range(U):
            s = src_ref[0, 0, base + k]
            d = dst_ref[0, 0, base + k]
            w0 = w0_ref[0, 0, base + k]
            w1 = w1_ref[0, 0, base + k]
            row = xw_ref[pl.ds(s, 1), :]
            wrow = jnp.concatenate(
                [jnp.full((1, OUT), w0, jnp.float32),
                 jnp.full((1, OUT), w1, jnp.float32)], axis=1)
            ds_.append(d)
            msgs.append(wrow * row)
            curs.append(out_ref[pl.ds(d, 1), :])
        vals = []
        for k in range(U):
            base_val = curs[k]
            for j in range(k):
                base_val = jnp.where(ds_[k] == ds_[j], vals[j], base_val)
            vals.append(base_val + msgs[k])
        for k in range(U):
            out_ref[pl.ds(ds_[k], 1), :] = vals[k]
        return 0

    lax.fori_loop(0, EB // U, body, 0)


def _selfbn_body(agg_ref, xw_ref, ws0_ref, ws1_ref, bias_ref,
                 pre_ref, psum_ref, psq_ref):
    pid = pl.program_id(0)

    @pl.when(pid == 0)
    def _():
        psum_ref[...] = jnp.zeros((8, H * OUT), jnp.float32)
        psq_ref[...] = jnp.zeros((8, H * OUT), jnp.float32)

    xwb = xw_ref[...]
    selfc = jnp.concatenate(
        [ws0_ref[:, 0:1] * xwb[:, :OUT], ws1_ref[:, 0:1] * xwb[:, OUT:]], axis=1)
    pre = agg_ref[...] + selfc + bias_ref[...]
    rid = pid * BD + lax.broadcasted_iota(jnp.int32, (BD, 1), 0)
    pre = pre * (rid < N).astype(jnp.float32)
    pre_ref[...] = pre
    psum_ref[0:1, :] = psum_ref[0:1, :] + jnp.sum(pre, axis=0, keepdims=True)
    psq_ref[0:1, :] = psq_ref[0:1, :] + jnp.sum(pre * pre, axis=0, keepdims=True)


def _bn_body(pre_ref, psum_ref, psq_ref, gamma_ref, beta_ref, y_ref):
    inv_n = 1.0 / N
    mean = psum_ref[0:1, :] * inv_n
    var = psq_ref[0:1, :] * inv_n - mean * mean
    rstd = lax.rsqrt(var + 1e-5)
    y = (pre_ref[...] - mean) * rstd * gamma_ref[...] + beta_ref[...]
    y_ref[...] = jnp.maximum(y, 0.0)


def kernel(x, edge_index, embedding, W, att_i, att_j, att_em_i, att_em_j,
           bias, gamma, beta):
    f32 = jnp.float32
    e = edge_index.shape[1]
    neb = -(-e // EB)
    epad = neb * EB
    src = edge_index[0].astype(jnp.int32)
    dst = edge_index[1].astype(jnp.int32)
    mask = src != dst
    dst_eff = jnp.where(mask, dst, DUMMY)
    src_p = jnp.concatenate([src, jnp.zeros((epad - e,), jnp.int32)])
    dst_p = jnp.concatenate([dst_eff, jnp.full((epad - e,), DUMMY, jnp.int32)])

    x_p = jnp.zeros((NPAD, IN), f32).at[:N].set(x)
    emb_p = jnp.zeros((NPAD, OUT), f32).at[:N].set(embedding)

    at = jnp.zeros((H * OUT, 128), f32)
    at = at.at[:OUT, 0].set(att_i[0, 0])
    at = at.at[OUT:, 1].set(att_i[0, 1])
    at = at.at[:OUT, 2].set(att_j[0, 0])
    at = at.at[OUT:, 3].set(att_j[0, 1])
    et = jnp.zeros((IN, 128), f32)
    et = et.at[:, 0].set(att_em_i[0, 0])
    et = et.at[:, 1].set(att_em_i[0, 1])
    et = et.at[:, 2].set(att_em_j[0, 0])
    et = et.at[:, 3].set(att_em_j[0, 1])

    xw, asc = pl.pallas_call(
        _dense_body,
        grid=(NPAD // BA,),
        in_specs=[
            pl.BlockSpec((BA, IN), lambda i: (i, 0)),
            pl.BlockSpec((IN, H * OUT), lambda i: (0, 0)),
            pl.BlockSpec((H * OUT, 128), lambda i: (0, 0)),
            pl.BlockSpec((IN, 128), lambda i: (0, 0)),
            pl.BlockSpec((BA, OUT), lambda i: (i, 0)),
        ],
        out_specs=[
            pl.BlockSpec((BA, H * OUT), lambda i: (i, 0)),
            pl.BlockSpec((BA, 128), lambda i: (i, 0)),
        ],
        out_shape=[
            jax.ShapeDtypeStruct((NPAD, H * OUT), f32),
            jax.ShapeDtypeStruct((NPAD, 128), f32),
        ],
    )(x_p, W.astype(f32), at, et, emb_p)

    ad0 = asc[:, 0]
    ad1 = asc[:, 1]
    as0 = asc[:, 2]
    as1 = asc[:, 3]

    ex0, ex1, sx0, sx1, p0, p1 = _make_sc_b1(epad)(
        src_p, dst_p, ad0, ad1, as0, as1)

    es0_8, es1_8 = pl.pallas_call(
        _reduce_body,
        grid=(1,),
        in_specs=[
            pl.BlockSpec((NW, NPAD), lambda i: (0, 0)),
            pl.BlockSpec((NW, NPAD), lambda i: (0, 0)),
        ],
        out_specs=[
            pl.BlockSpec((8, NPAD), lambda i: (0, 0)),
            pl.BlockSpec((8, NPAD), lambda i: (0, 0)),
        ],
        out_shape=[
            jax.ShapeDtypeStruct((8, NPAD), f32),
            jax.ShapeDtypeStruct((8, NPAD), f32),
        ],
    )(p0, p1)
    es0 = es0_8[0]
    es1 = es1_8[0]

    w0, w1, ws0, ws1 = _make_sc_b3(epad)(dst_p, ex0, ex1, sx0, sx1, es0, es1)

    smem = pltpu.MemorySpace.SMEM
    agg = pl.pallas_call(
        _agg_body,
        grid=(neb,),
        in_specs=[
            pl.BlockSpec((1, 1, EB), lambda i: (i, 0, 0), memory_space=smem),
            pl.BlockSpec((1, 1, EB), lambda i: (i, 0, 0), memory_space=smem),
            pl.BlockSpec((1, 1, EB), lambda i: (i, 0, 0), memory_space=smem),
            pl.BlockSpec((1, 1, EB), lambda i: (i, 0, 0), memory_space=smem),
            pl.BlockSpec((NPAD, H * OUT), lambda i: (0, 0)),
        ],
        out_specs=pl.BlockSpec((NPAD, H * OUT), lambda i: (0, 0)),
        out_shape=jax.ShapeDtypeStruct((NPAD, H * OUT), f32),
    )(src_p.reshape(neb, 1, EB), dst_p.reshape(neb, 1, EB),
      w0.reshape(neb, 1, EB), w1.reshape(neb, 1, EB), xw)

    ws0b = jnp.broadcast_to(ws0[:, None], (NPAD, 128))
    ws1b = jnp.broadcast_to(ws1[:, None], (NPAD, 128))
    bias2 = bias.astype(f32).reshape(1, H * OUT)
    pre, psum, psq = pl.pallas_call(
        _selfbn_body,
        grid=(NPAD // BD,),
        in_specs=[
            pl.BlockSpec((BD, H * OUT), lambda i: (i, 0)),
            pl.BlockSpec((BD, H * OUT), lambda i: (i, 0)),
            pl.BlockSpec((BD, 128), lambda i: (i, 0)),
            pl.BlockSpec((BD, 128), lambda i: (i, 0)),
            pl.BlockSpec((1, H * OUT), lambda i: (0, 0)),
        ],
        out_specs=[
            pl.BlockSpec((BD, H * OUT), lambda i: (i, 0)),
            pl.BlockSpec((8, H * OUT), lambda i: (0, 0)),
            pl.BlockSpec((8, H * OUT), lambda i: (0, 0)),
        ],
        out_shape=[
            jax.ShapeDtypeStruct((NPAD, H * OUT), f32),
            jax.ShapeDtypeStruct((8, H * OUT), f32),
            jax.ShapeDtypeStruct((8, H * OUT), f32),
        ],
    )(agg, xw, ws0b, ws1b, bias2)

    y = pl.pallas_call(
        _bn_body,
        grid=(NPAD // BD,),
        in_specs=[
            pl.BlockSpec((BD, H * OUT), lambda i: (i, 0)),
            pl.BlockSpec((8, H * OUT), lambda i: (0, 0)),
            pl.BlockSpec((8, H * OUT), lambda i: (0, 0)),
            pl.BlockSpec((1, H * OUT), lambda i: (0, 0)),
            pl.BlockSpec((1, H * OUT), lambda i: (0, 0)),
        ],
        out_specs=pl.BlockSpec((BD, H * OUT), lambda i: (i, 0)),
        out_shape=jax.ShapeDtypeStruct((NPAD, H * OUT), f32),
    )(pre, psum, psq, gamma.astype(f32).reshape(1, H * OUT),
      beta.astype(f32).reshape(1, H * OUT))

    return y[:N]
